# baseline (speedup 1.0000x reference)
import functools

import jax
import jax.numpy as jnp
from jax import lax
from jax.experimental import pallas as pl
from jax.experimental.pallas import tpu as pltpu
from jax.experimental.pallas import tpu_sc as plsc

_B = 4
_S = 4096
_D = 1024
_N = _B * _S
_NC = 2
_NS = 16
_NW = _NC * _NS
_PW = _S // _NW
_C = 16
_NCHUNK = _PW // _C
_L = 16

_BCAST_DNUMS = lax.GatherDimensionNumbers(offset_dims=(),
                                          collapsed_slice_dims=(0,),
                                          start_index_map=(0,))


def _sc_kernel(tok_hbm, w_hbm, out_hbm, tok_v, tbuf0, tbuf1, wbuf0, wbuf1,
               rsem0, rsem1, osem0, osem1):
    wid = lax.axis_index("s") * _NC + lax.axis_index("c")
    p0 = wid * _PW

    for b in range(_B):
        pltpu.sync_copy(tok_hbm.at[pl.ds(b * _S + p0, _PW)],
                        tok_v.at[pl.ds(b * _PW, _PW)])

    tbufs = (tbuf0, tbuf1)
    rsems = (rsem0, rsem1)
    wbufs = (wbuf0, wbuf1)
    osems = (osem0, osem1)

    def read(c):
        return pltpu.async_copy(w_hbm.at[pl.ds(p0 + c * _C + 1, _C)],
                                tbufs[c % 2], rsems[c % 2])

    def mask_rows(tbuf, wbuf, mval):
        def row(r, _):
            ridx = (lax.iota(jnp.int32, _L) * 0 + r).reshape(_L, 1)
            mrow = lax.gather(mval, ridx, _BCAST_DNUMS, (1,),
                              mode=lax.GatherScatterMode.PROMISE_IN_BOUNDS)

            def sub(ss, _):
                for kk in range(_D // 8 // _L):
                    wbuf[r, ss, pl.ds(kk * _L, _L)] = (
                        tbuf[r, ss, pl.ds(kk * _L, _L)] * mrow)
                return 0

            lax.fori_loop(0, 8, sub, 0)
            return 0

        lax.fori_loop(0, _C, row, 0)

    pending = [read(0), read(1)]
    out_pending = [None, None]
    t = 0
    for c in range(_NCHUNK):
        p = c % 2
        pending[p].wait()
        for b in range(_B):
            q = t % 2
            t += 1
            tok = tok_v[pl.ds(b * _PW + c * _C, _L)]
            mval = jnp.where(tok != 0, 1.0, 0.0)
            if out_pending[q] is not None:
                out_pending[q].wait()
            mask_rows(tbufs[p], wbufs[q], mval)
            out_pending[q] = pltpu.async_copy(
                wbufs[q], out_hbm.at[pl.ds(b * _S + p0 + c * _C, _C)],
                osems[q])
        if c + 2 < _NCHUNK:
            pending[p] = read(c + 2)
    out_pending[0].wait()
    out_pending[1].wait()


@jax.jit
def _run(tok_flat, weights):
    mesh = plsc.VectorSubcoreMesh(core_axis_name="c", subcore_axis_name="s")
    f = functools.partial(
        pl.kernel,
        mesh=mesh,
        out_type=jax.ShapeDtypeStruct((_N, 8, _D // 8), jnp.float32),
        scratch_types=[
            pltpu.VMEM((_B * _PW,), jnp.int32),
            pltpu.VMEM((_C, 8, _D // 8), jnp.float32),
            pltpu.VMEM((_C, 8, _D // 8), jnp.float32),
            pltpu.VMEM((_C, 8, _D // 8), jnp.float32),
            pltpu.VMEM((_C, 8, _D // 8), jnp.float32),
            pltpu.SemaphoreType.DMA,
            pltpu.SemaphoreType.DMA,
            pltpu.SemaphoreType.DMA,
            pltpu.SemaphoreType.DMA,
        ],
    )(_sc_kernel)
    return f(tok_flat, weights)


def kernel(input, weights):
    tok_flat = input.reshape(-1)
    out = _run(tok_flat, weights.reshape(-1, 8, _D // 8))
    return out.reshape(_B, _S, _D)


# speedup vs baseline: 4.1472x; 4.1472x over previous
import functools

import jax
import jax.numpy as jnp
from jax import lax
from jax.experimental import pallas as pl
from jax.experimental.pallas import tpu as pltpu
from jax.experimental.pallas import tpu_sc as plsc

_B = 4
_S = 4096
_D = 1024
_N = _B * _S
_NC = 2
_NS = 16
_NW = _NC * _NS
_RW = _N // _NW
_C = 32
_NCHUNK = _RW // _C
_NB = 3
_L = 16


def _sc_kernel(tok_hbm, w_hbm, out_hbm, tok_v, idx_v, buf0, buf1, buf2,
               gsem0, gsem1, gsem2, osem0, osem1, osem2):
    wid = lax.axis_index("s") * _NC + lax.axis_index("c")
    base = wid * _RW
    pos0 = lax.rem(base, _S) + 1

    pltpu.sync_copy(tok_hbm.at[pl.ds(base, _RW)], tok_v)

    def build_idx(j, _):
        tok = tok_v[pl.ds(j * _L, _L)]
        ramp = lax.iota(jnp.int32, _L) + (pos0 + j * _L)
        idx_v[pl.ds(j * _L, _L)] = jnp.where(tok != 0, ramp, 0)
        return 0

    lax.fori_loop(0, _RW // _L, build_idx, 0)

    bufs = (buf0, buf1, buf2)
    gsems = (gsem0, gsem1, gsem2)
    osems = (osem0, osem1, osem2)

    def gather(c):
        p = c % _NB
        return pltpu.async_copy(w_hbm.at[idx_v.at[pl.ds(c * _C, _C)]],
                                bufs[p], gsems[p])

    pending = [gather(c) for c in range(_NB)]
    writes = [None] * _NCHUNK
    for c in range(_NCHUNK):
        p = c % _NB
        pending[p].wait()
        writes[c] = pltpu.async_copy(
            bufs[p], out_hbm.at[pl.ds(base + c * _C, _C)], osems[p])
        if c >= 1 and c + 2 < _NCHUNK:
            writes[c - 1].wait()
            writes[c - 1] = None
            pending[(c + 2) % _NB] = gather(c + 2)
    for wcp in writes:
        if wcp is not None:
            wcp.wait()


@jax.jit
def _run(tok_flat, weights):
    mesh = plsc.VectorSubcoreMesh(core_axis_name="c", subcore_axis_name="s")
    f = functools.partial(
        pl.kernel,
        mesh=mesh,
        out_type=jax.ShapeDtypeStruct((_N, _D), jnp.float32),
        scratch_types=[
            pltpu.VMEM((_RW,), jnp.int32),
            pltpu.VMEM((_RW,), jnp.int32),
            pltpu.VMEM((_C, _D), jnp.float32),
            pltpu.VMEM((_C, _D), jnp.float32),
            pltpu.VMEM((_C, _D), jnp.float32),
            pltpu.SemaphoreType.DMA,
            pltpu.SemaphoreType.DMA,
            pltpu.SemaphoreType.DMA,
            pltpu.SemaphoreType.DMA,
            pltpu.SemaphoreType.DMA,
            pltpu.SemaphoreType.DMA,
        ],
    )(_sc_kernel)
    return f(tok_flat, weights)


def kernel(input, weights):
    tok_flat = input.reshape(-1)
    out = _run(tok_flat, weights)
    return out.reshape(_B, _S, _D)
